# Initial kernel scaffold; baseline (speedup 1.0000x reference)
import jax
import jax.numpy as jnp
from jax.experimental import pallas as pl


def kernel(x, edge_index, batch, edge_attr, PH1_feat, PH0_feat, params, vae_eps):
    raise NotImplementedError("write your pallas kernel here")


# speedup vs baseline: 15.5657x; 15.5657x over previous
import functools

import jax
import jax.numpy as jnp
from jax import lax
from jax.experimental import pallas as pl
from jax.experimental.pallas import tpu as pltpu
from jax.experimental.pallas import tpu_sc as plsc

N = 10000
NPAD = 10240
E = 320000
B = 64
INFEAT = 128
NHID = 64
OUTF = 16

NC = 2
NS = 16
NW = NC * NS
EPW = E // NW
K = 80
NCHUNK = EPW // K
RPT = NPAD // NS

_MESH = dict(core_axis_name="c", subcore_axis_name="s", num_cores=NC,
             num_subcores=NS)
_SC_PARAMS = pltpu.CompilerParams(needs_layout_passes=False,
                                  use_tc_tiling_on_sc=False)

_NEG_INF = float("-inf")
_BN_SCALE = 1.0 / (1.0 + 1e-5) ** 0.5


def _iota16():
    return lax.broadcasted_iota(jnp.int32, (16,), 0)


def _zeros16(dtype=jnp.float32):
    return jnp.zeros((16,), dtype)


def _deg_body(col_hbm, w_hbm, deg_hbm, col_v, w_v, zb, deg_s):
    cid = lax.axis_index("c")
    sid = lax.axis_index("s")
    wid = sid * NC + cid
    pltpu.sync_copy(col_hbm.at[wid], col_v)
    pltpu.sync_copy(w_hbm.at[wid], w_v)

    def zrow(r, c):
        zb[pl.ds(r * 16, 16)] = _zeros16()
        return c
    lax.fori_loop(0, RPT // 16, zrow, 0)
    pltpu.sync_copy(zb, deg_s.at[pl.ds(sid * RPT, RPT)])
    plsc.subcore_barrier()

    def chunk(i, c):
        pltpu.sync_copy(w_v.at[pl.ds(i * K, K)], deg_s.at[col_v.at[i]],
                        add=True)
        return c
    lax.fori_loop(0, NCHUNK, chunk, 0)
    plsc.subcore_barrier()
    pltpu.sync_copy(deg_s.at[pl.ds(sid * RPT, RPT)],
                    deg_hbm.at[pl.ds(cid * NPAD + sid * RPT, RPT)])


@functools.cache
def _deg_call():
    return pl.kernel(
        _deg_body,
        out_type=jax.ShapeDtypeStruct((2 * NPAD,), jnp.float32),
        mesh=plsc.VectorSubcoreMesh(**_MESH),
        compiler_params=_SC_PARAMS,
        scratch_types=[
            pltpu.VMEM((NCHUNK, K), jnp.int32),
            pltpu.VMEM((EPW,), jnp.float32),
            pltpu.VMEM((RPT,), jnp.float32),
            pltpu.VMEM_SHARED((NPAD,), jnp.float32),
        ],
    )


def _mp_body(F, hs_hbm, row_hbm, col_hbm, w_hbm, dinv_hbm, out_hbm,
             row_v, col_v, w_v, s_v, dinv_v, rb, acc_s, gsem):
    G = F // 16
    cid = lax.axis_index("c")
    sid = lax.axis_index("s")
    wid = sid * NC + cid
    pltpu.sync_copy(row_hbm.at[wid], row_v)
    pltpu.sync_copy(col_hbm.at[wid], col_v)
    pltpu.sync_copy(w_hbm.at[wid], w_v)
    pltpu.sync_copy(dinv_hbm, dinv_v)

    def zrow(r, c):
        for g in range(G):
            rb[r, pl.ds(g * 16, 16)] = _zeros16()
        return c
    lax.fori_loop(0, K, zrow, 0)
    for r8 in range(RPT // K):
        pltpu.sync_copy(rb, acc_s.at[pl.ds(sid * RPT + r8 * K, K)])
    plsc.subcore_barrier()

    def srow(i, c):
        c16 = col_v[i // (K // 16), pl.ds((i % (K // 16)) * 16, 16)]
        d16 = plsc.load_gather(dinv_v, [c16])
        s_v[pl.ds(i * 16, 16)] = w_v[pl.ds(i * 16, 16)] * d16
        return c
    lax.fori_loop(0, EPW // 16, srow, 0)

    def chunk(i, c):
        pltpu.async_copy(hs_hbm.at[row_v.at[pl.ds(i * K, K)]], rb, gsem).wait()

        def scale(e, c2):
            bc = plsc.load_gather(s_v, [jnp.full((16,), i * K + e, jnp.int32)])
            for g in range(G):
                rb[e, pl.ds(g * 16, 16)] = rb[e, pl.ds(g * 16, 16)] * bc
            return c2
        lax.fori_loop(0, K, scale, 0)
        pltpu.sync_copy(rb, acc_s.at[col_v.at[i]], add=True)
        return c
    lax.fori_loop(0, NCHUNK, chunk, 0)
    plsc.subcore_barrier()
    pltpu.sync_copy(acc_s.at[pl.ds(sid * RPT, RPT)],
                    out_hbm.at[pl.ds(cid * NPAD + sid * RPT, RPT)])


@functools.cache
def _make_mp_call(F):
    return pl.kernel(
        functools.partial(_mp_body, F),
        out_type=jax.ShapeDtypeStruct((2 * NPAD, F), jnp.float32),
        mesh=plsc.VectorSubcoreMesh(**_MESH),
        compiler_params=_SC_PARAMS,
        scratch_types=[
            pltpu.VMEM((EPW,), jnp.int32),
            pltpu.VMEM((NCHUNK, K), jnp.int32),
            pltpu.VMEM((EPW,), jnp.float32),
            pltpu.VMEM((EPW,), jnp.float32),
            pltpu.VMEM((NPAD,), jnp.float32),
            pltpu.VMEM((K, F), jnp.float32),
            pltpu.VMEM_SHARED((NPAD, F), jnp.float32),
            pltpu.SemaphoreType.DMA,
        ],
    )


def _matT(a, w):
    return lax.dot_general(a, w, (((1,), (1,)), ((), ())),
                           preferred_element_type=jnp.float32)


def _lrelu(x):
    return jnp.where(x > 0, x, 0.01 * x)


def _tc1_body(x_ref, w1_ref, deg0_ref, deg1_ref, ph1_ref, ph0_ref, eps_ref,
              e0w_ref, e0b_ref, e1w_ref, e1b_ref, d0w_ref, d0b_ref,
              d1w_ref, d1b_ref, ct0w_ref, ct0b_ref, bn1g_ref, bn1b_ref,
              bn2g_ref, bn2b_ref, ct1w_ref, ct1b_ref,
              hs_ref, dinv_ref, mu_ref, land_ref, ldec_ref, betti_ref):
    deg = 1.0 + deg0_ref[...] + deg1_ref[...]
    dinv = lax.rsqrt(deg)
    dinv_ref[...] = dinv
    h0 = _matT(x_ref[...], w1_ref[...])
    hs_ref[...] = h0 * dinv[:N]

    emb = _lrelu(_matT(ph1_ref[...], e0w_ref[...]) + e0b_ref[...])
    mu = _matT(emb, e1w_ref[...]) + e1b_ref[...]
    mu_ref[...] = mu
    std = jnp.exp(0.5 * mu)
    land = eps_ref[...] * std + mu
    land_ref[...] = land
    dec = _lrelu(_matT(land, d0w_ref[...]) + d0b_ref[...])
    ld = _matT(dec, d1w_ref[...]) + d1b_ref[...]
    ldec_ref[...] = 1.0 / (1.0 + jnp.exp(-ld))

    bc = _matT(ph0_ref[...], ct0w_ref[...]) + ct0b_ref[...]
    bc = bc * _BN_SCALE * bn1g_ref[...] + bn1b_ref[...]
    bc = _lrelu(bc)
    bc = bc * _BN_SCALE * bn2g_ref[...] + bn2b_ref[...]
    betti_ref[...] = _matT(bc, ct1w_ref[...]) + ct1b_ref[...]


def _seg_mean(h, batch):
    maskf = (batch == lax.broadcasted_iota(jnp.int32, (N, B), 1)
             ).astype(jnp.float32)
    sums = lax.dot_general(maskf, h, (((0,), (0,)), ((), ())),
                           preferred_element_type=jnp.float32)
    cnt = lax.dot_general(maskf, jnp.ones((N, 1), jnp.float32),
                          (((0,), (0,)), ((), ())),
                          preferred_element_type=jnp.float32)
    return sums / jnp.maximum(cnt, 1.0)


def _maxpool_body(h_ref, batch_ref, out_ref):
    b = pl.program_id(0)
    mask = batch_ref[...] == b
    out_ref[pl.ds(b, 1), :] = jnp.max(
        jnp.where(mask, h_ref[...], _NEG_INF), axis=0, keepdims=True)


def _maxpool(h, batch2, F):
    return pl.pallas_call(
        _maxpool_body,
        grid=(B,),
        in_specs=[
            pl.BlockSpec((N, F), lambda b: (0, 0)),
            pl.BlockSpec((N, 1), lambda b: (0, 0)),
        ],
        out_specs=pl.BlockSpec((B, F), lambda b: (0, 0)),
        out_shape=jax.ShapeDtypeStruct((B, F), jnp.float32),
    )(h, batch2)


def _tc2_body(a0_ref, a1_ref, hs_ref, dinv_ref, b1_ref, w2_ref, batch_ref,
              h1_ref, x1mean_ref, h1s_ref):
    dinv = dinv_ref[...]
    h1 = a0_ref[...] + a1_ref[...] + dinv * hs_ref[...] + b1_ref[...]
    h1 = jnp.maximum(h1, 0.0)
    h1_ref[...] = h1
    x1mean_ref[...] = _seg_mean(h1, batch_ref[...])
    h1s_ref[...] = _matT(h1, w2_ref[...]) * dinv


def _tc3a_body(a0_ref, a1_ref, h1s_ref, dinv_ref, b2_ref, batch_ref,
               h2_ref, x2mean_ref):
    h2 = (a0_ref[...] + a1_ref[...] + dinv_ref[...] * h1s_ref[...]
          + b2_ref[...])
    h2_ref[...] = h2
    x2mean_ref[...] = _seg_mean(h2, batch_ref[...])


def _tc3_body(x1max_ref, x1mean_ref, x2max_ref, x2mean_ref,
              land_ref, betti_ref,
              m1wa_ref, m1wb_ref, m1wc_ref, m1wd_ref, m1we_ref, m1wf_ref,
              g0a_ref, g0b_ref, g0c_ref, g0d_ref, g0e_ref, g0f_ref,
              b0a_ref, b0b_ref, b0c_ref, b0d_ref, b0e_ref, b0f_ref,
              m1b_ref, bn1g_ref, bn1b_ref, m2w_ref, m2b_ref,
              clw_ref, clb_ref, cls_ref):
    pieces = [
        (x1max_ref[...], m1wa_ref, g0a_ref, b0a_ref),
        (x1mean_ref[...], m1wb_ref, g0b_ref, b0b_ref),
        (x2max_ref[...], m1wc_ref, g0c_ref, b0c_ref),
        (x2mean_ref[...], m1wd_ref, g0d_ref, b0d_ref),
        (land_ref[...], m1we_ref, g0e_ref, b0e_ref),
        (betti_ref[...], m1wf_ref, g0f_ref, b0f_ref),
    ]
    f = m1b_ref[...]
    for val, wref, gref, bref in pieces:
        v = val * _BN_SCALE * gref[...] + bref[...]
        f = f + _matT(v, wref[...])
    f = jnp.maximum(f, 0.0)
    f = f * _BN_SCALE * bn1g_ref[...] + bn1b_ref[...]
    f = jnp.maximum(_matT(f, m2w_ref[...]) + m2b_ref[...], 0.0)
    logits = _matT(f, clw_ref[...]) + clb_ref[...]
    m = jnp.max(logits, axis=1, keepdims=True)
    lse = m + jnp.log(jnp.sum(jnp.exp(logits - m), axis=1, keepdims=True))
    cls_ref[...] = logits - lse


def _row2(v):
    return v.reshape(1, -1)


def kernel(x, edge_index, batch, edge_attr, PH1_feat, PH0_feat, params,
           vae_eps):
    p = params
    row = edge_index[0].reshape(NW, EPW)
    col = edge_index[1].reshape(NW, NCHUNK, K)
    w = edge_attr.reshape(NW, EPW)

    deg2 = _deg_call()(col, w)
    deg0 = deg2[:NPAD].reshape(NPAD, 1)
    deg1 = deg2[NPAD:].reshape(NPAD, 1)

    tc1 = pl.pallas_call(
        _tc1_body,
        out_shape=[
            jax.ShapeDtypeStruct((N, NHID), jnp.float32),
            jax.ShapeDtypeStruct((NPAD, 1), jnp.float32),
            jax.ShapeDtypeStruct((B, 16), jnp.float32),
            jax.ShapeDtypeStruct((B, 16), jnp.float32),
            jax.ShapeDtypeStruct((B, 1000), jnp.float32),
            jax.ShapeDtypeStruct((B, 32), jnp.float32),
        ],
    )
    hs, dinv, mu, land, ldec, betti = tc1(
        x, p['c1_W'], deg0, deg1, PH1_feat, PH0_feat, vae_eps,
        p['e0_W'], _row2(p['e0_b']), p['e1_W'], _row2(p['e1_b']),
        p['d0_W'], _row2(p['d0_b']), p['d1_W'], _row2(p['d1_b']),
        p['ct0_W'], _row2(p['ct0_b']), _row2(p['ct_bn1_g']),
        _row2(p['ct_bn1_b']), _row2(p['ct_bn2_g']), _row2(p['ct_bn2_b']),
        p['ct1_W'], _row2(p['ct1_b']))

    dinv_flat = dinv.reshape(NPAD)
    acc1 = _make_mp_call(NHID)(hs, row, col, w, dinv_flat)

    tc2 = pl.pallas_call(
        _tc2_body,
        out_shape=[
            jax.ShapeDtypeStruct((N, NHID), jnp.float32),
            jax.ShapeDtypeStruct((B, NHID), jnp.float32),
            jax.ShapeDtypeStruct((N, OUTF), jnp.float32),
        ],
    )
    batch2 = batch.reshape(N, 1)
    h1, x1mean, h1s = tc2(acc1[:N], acc1[NPAD:NPAD + N], hs, dinv[:N],
                          _row2(p['c1_b']), p['c2_W'], batch2)
    x1max = _maxpool(h1, batch2, NHID)

    acc2 = _make_mp_call(OUTF)(h1s, row, col, w, dinv_flat)

    tc3a = pl.pallas_call(
        _tc3a_body,
        out_shape=[
            jax.ShapeDtypeStruct((N, OUTF), jnp.float32),
            jax.ShapeDtypeStruct((B, OUTF), jnp.float32),
        ],
    )
    h2, x2mean = tc3a(acc2[:N], acc2[NPAD:NPAD + N], h1s, dinv[:N],
                      _row2(p['c2_b']), batch2)
    x2max = _maxpool(h2, batch2, OUTF)

    m1 = p['m1_W']
    g0 = p['m_bn0_g']
    b0 = p['m_bn0_b']
    splits = [0, 64, 128, 144, 160, 176, 208]
    m1w = [m1[:, splits[i]:splits[i + 1]] for i in range(6)]
    g0s = [_row2(g0[splits[i]:splits[i + 1]]) for i in range(6)]
    b0s = [_row2(b0[splits[i]:splits[i + 1]]) for i in range(6)]

    tc3 = pl.pallas_call(
        _tc3_body,
        out_shape=[jax.ShapeDtypeStruct((B, 2), jnp.float32)],
    )
    (cls,) = tc3(x1max, x1mean, x2max, x2mean, land, betti,
                 *m1w, *g0s, *b0s,
                 _row2(p['m1_b']), _row2(p['m_bn1_g']), _row2(p['m_bn1_b']),
                 p['m2_W'], _row2(p['m2_b']), p['cl_W'], _row2(p['cl_b']))

    return (cls, mu, mu, land, ldec, betti)


# speedup vs baseline: 21.4334x; 1.3770x over previous
import functools

import jax
import jax.numpy as jnp
from jax import lax
from jax.experimental import pallas as pl
from jax.experimental.pallas import tpu as pltpu
from jax.experimental.pallas import tpu_sc as plsc

N = 10000
NPAD = 10240
E = 320000
B = 64
INFEAT = 128
NHID = 64
OUTF = 16

NC = 2
NS = 16
NW = NC * NS
EPW = E // NW
K = 80
NCHUNK = EPW // K
RPT = NPAD // NS

_MESH = dict(core_axis_name="c", subcore_axis_name="s", num_cores=NC,
             num_subcores=NS)
_SC_PARAMS = pltpu.CompilerParams(needs_layout_passes=False,
                                  use_tc_tiling_on_sc=False)

_NEG_INF = float("-inf")
_BN_SCALE = 1.0 / (1.0 + 1e-5) ** 0.5


def _iota16():
    return lax.broadcasted_iota(jnp.int32, (16,), 0)


def _zeros16(dtype=jnp.float32):
    return jnp.zeros((16,), dtype)


def _deg_body(col_hbm, w_hbm, deg_hbm, col_v, w_v, zb, deg_s):
    cid = lax.axis_index("c")
    sid = lax.axis_index("s")
    wid = sid * NC + cid
    pltpu.sync_copy(col_hbm.at[wid], col_v)
    pltpu.sync_copy(w_hbm.at[wid], w_v)

    def zrow(r, c):
        zb[pl.ds(r * 16, 16)] = _zeros16()
        return c
    lax.fori_loop(0, RPT // 16, zrow, 0)
    pltpu.sync_copy(zb, deg_s.at[pl.ds(sid * RPT, RPT)])
    plsc.subcore_barrier()

    def chunk(i, c):
        pltpu.sync_copy(w_v.at[pl.ds(i * K, K)], deg_s.at[col_v.at[i]],
                        add=True)
        return c
    lax.fori_loop(0, NCHUNK, chunk, 0)
    plsc.subcore_barrier()
    pltpu.sync_copy(deg_s.at[pl.ds(sid * RPT, RPT)],
                    deg_hbm.at[pl.ds(cid * NPAD + sid * RPT, RPT)])


@functools.cache
def _deg_call():
    return pl.kernel(
        _deg_body,
        out_type=jax.ShapeDtypeStruct((2 * NPAD,), jnp.float32),
        mesh=plsc.VectorSubcoreMesh(**_MESH),
        compiler_params=_SC_PARAMS,
        scratch_types=[
            pltpu.VMEM((NCHUNK, K), jnp.int32),
            pltpu.VMEM((EPW,), jnp.float32),
            pltpu.VMEM((RPT,), jnp.float32),
            pltpu.VMEM_SHARED((NPAD,), jnp.float32),
        ],
    )


def _mp_body(F, hs_hbm, row_hbm, col_hbm, w_hbm, dinv_hbm, out_hbm,
             row_v, col_v, w_v, s_v, dinv_v, rb0, rb1, acc_s, sem0, sem1):
    G = F // 16
    U = 4
    cid = lax.axis_index("c")
    sid = lax.axis_index("s")
    wid = sid * NC + cid
    pltpu.sync_copy(row_hbm.at[wid], row_v)
    pltpu.sync_copy(col_hbm.at[wid], col_v)
    pltpu.sync_copy(w_hbm.at[wid], w_v)
    pltpu.sync_copy(dinv_hbm, dinv_v)

    def zrow(r, c):
        for g in range(G):
            rb0[r, pl.ds(g * 16, 16)] = _zeros16()
        return c
    lax.fori_loop(0, K, zrow, 0)
    for r8 in range(RPT // K):
        pltpu.sync_copy(rb0, acc_s.at[pl.ds(sid * RPT + r8 * K, K)])
    plsc.subcore_barrier()

    pltpu.async_copy(hs_hbm.at[row_v.at[pl.ds(0, K)]], rb0, sem0)

    def srow(i, c):
        c16 = col_v[i // (K // 16), pl.ds((i % (K // 16)) * 16, 16)]
        d16 = plsc.load_gather(dinv_v, [c16])
        s_v[pl.ds(i * 16, 16)] = w_v[pl.ds(i * 16, 16)] * d16
        return c
    lax.fori_loop(0, EPW // 16, srow, 0)

    def scale_scatter(i, buf):
        def scale(t, c2):
            e = t * U
            for u in range(U):
                bc = plsc.load_gather(
                    s_v, [jnp.full((16,), i * K + e + u, jnp.int32)])
                for g in range(G):
                    buf[e + u, pl.ds(g * 16, 16)] = (
                        buf[e + u, pl.ds(g * 16, 16)] * bc)
            return c2
        lax.fori_loop(0, K // U, scale, 0)
        pltpu.sync_copy(buf, acc_s.at[col_v.at[i]], add=True)

    def pair(g, c):
        i0 = 2 * g
        pltpu.make_async_copy(hs_hbm.at[pl.ds(0, K)], rb0, sem0).wait()
        pltpu.async_copy(hs_hbm.at[row_v.at[pl.ds((i0 + 1) * K, K)]], rb1,
                         sem1)
        scale_scatter(i0, rb0)
        pltpu.make_async_copy(hs_hbm.at[pl.ds(0, K)], rb1, sem1).wait()
        pltpu.async_copy(hs_hbm.at[row_v.at[pl.ds((i0 + 2) * K, K)]], rb0,
                         sem0)
        scale_scatter(i0 + 1, rb1)
        return c
    lax.fori_loop(0, (NCHUNK - 1) // 2, pair, 0)
    pltpu.make_async_copy(hs_hbm.at[pl.ds(0, K)], rb0, sem0).wait()
    scale_scatter(NCHUNK - 1, rb0)

    plsc.subcore_barrier()
    pltpu.sync_copy(acc_s.at[pl.ds(sid * RPT, RPT)],
                    out_hbm.at[pl.ds(cid * NPAD + sid * RPT, RPT)])


@functools.cache
def _make_mp_call(F):
    return pl.kernel(
        functools.partial(_mp_body, F),
        out_type=jax.ShapeDtypeStruct((2 * NPAD, F), jnp.float32),
        mesh=plsc.VectorSubcoreMesh(**_MESH),
        compiler_params=_SC_PARAMS,
        scratch_types=[
            pltpu.VMEM((EPW,), jnp.int32),
            pltpu.VMEM((NCHUNK, K), jnp.int32),
            pltpu.VMEM((EPW,), jnp.float32),
            pltpu.VMEM((EPW,), jnp.float32),
            pltpu.VMEM((NPAD,), jnp.float32),
            pltpu.VMEM((K, F), jnp.float32),
            pltpu.VMEM((K, F), jnp.float32),
            pltpu.VMEM_SHARED((NPAD, F), jnp.float32),
            pltpu.SemaphoreType.DMA,
            pltpu.SemaphoreType.DMA,
        ],
    )


def _matT(a, w):
    return lax.dot_general(a, w, (((1,), (1,)), ((), ())),
                           preferred_element_type=jnp.float32)


def _lrelu(x):
    return jnp.where(x > 0, x, 0.01 * x)


def _tc1_body(x_ref, w1_ref, deg0_ref, deg1_ref, ph1_ref, ph0_ref, eps_ref,
              e0w_ref, e0b_ref, e1w_ref, e1b_ref, d0w_ref, d0b_ref,
              d1w_ref, d1b_ref, ct0w_ref, ct0b_ref, bn1g_ref, bn1b_ref,
              bn2g_ref, bn2b_ref, ct1w_ref, ct1b_ref,
              hs_ref, dinv_ref, mu_ref, land_ref, ldec_ref, betti_ref):
    deg = 1.0 + deg0_ref[...] + deg1_ref[...]
    dinv = lax.rsqrt(deg)
    dinv_ref[...] = dinv
    h0 = _matT(x_ref[...], w1_ref[...])
    hs_ref[...] = h0 * dinv[:N]

    emb = _lrelu(_matT(ph1_ref[...], e0w_ref[...]) + e0b_ref[...])
    mu = _matT(emb, e1w_ref[...]) + e1b_ref[...]
    mu_ref[...] = mu
    std = jnp.exp(0.5 * mu)
    land = eps_ref[...] * std + mu
    land_ref[...] = land
    dec = _lrelu(_matT(land, d0w_ref[...]) + d0b_ref[...])
    ld = _matT(dec, d1w_ref[...]) + d1b_ref[...]
    ldec_ref[...] = 1.0 / (1.0 + jnp.exp(-ld))

    bc = _matT(ph0_ref[...], ct0w_ref[...]) + ct0b_ref[...]
    bc = bc * _BN_SCALE * bn1g_ref[...] + bn1b_ref[...]
    bc = _lrelu(bc)
    bc = bc * _BN_SCALE * bn2g_ref[...] + bn2b_ref[...]
    betti_ref[...] = _matT(bc, ct1w_ref[...]) + ct1b_ref[...]


def _seg_mean(h, batch):
    maskf = (batch == lax.broadcasted_iota(jnp.int32, (N, B), 1)
             ).astype(jnp.float32)
    sums = lax.dot_general(maskf, h, (((0,), (0,)), ((), ())),
                           preferred_element_type=jnp.float32)
    cnt = lax.dot_general(maskf, jnp.ones((N, 1), jnp.float32),
                          (((0,), (0,)), ((), ())),
                          preferred_element_type=jnp.float32)
    return sums / jnp.maximum(cnt, 1.0)


def _maxpool_body(h_ref, batch_ref, out_ref):
    b = pl.program_id(0)
    mask = batch_ref[...] == b
    out_ref[pl.ds(b, 1), :] = jnp.max(
        jnp.where(mask, h_ref[...], _NEG_INF), axis=0, keepdims=True)


def _maxpool(h, batch2, F):
    return pl.pallas_call(
        _maxpool_body,
        grid=(B,),
        in_specs=[
            pl.BlockSpec((N, F), lambda b: (0, 0)),
            pl.BlockSpec((N, 1), lambda b: (0, 0)),
        ],
        out_specs=pl.BlockSpec((B, F), lambda b: (0, 0)),
        out_shape=jax.ShapeDtypeStruct((B, F), jnp.float32),
    )(h, batch2)


def _tc2_body(a0_ref, a1_ref, hs_ref, dinv_ref, b1_ref, w2_ref, batch_ref,
              h1_ref, x1mean_ref, h1s_ref):
    dinv = dinv_ref[...]
    h1 = a0_ref[...] + a1_ref[...] + dinv * hs_ref[...] + b1_ref[...]
    h1 = jnp.maximum(h1, 0.0)
    h1_ref[...] = h1
    x1mean_ref[...] = _seg_mean(h1, batch_ref[...])
    h1s_ref[...] = _matT(h1, w2_ref[...]) * dinv


def _tc3a_body(a0_ref, a1_ref, h1s_ref, dinv_ref, b2_ref, batch_ref,
               h2_ref, x2mean_ref):
    h2 = (a0_ref[...] + a1_ref[...] + dinv_ref[...] * h1s_ref[...]
          + b2_ref[...])
    h2_ref[...] = h2
    x2mean_ref[...] = _seg_mean(h2, batch_ref[...])


def _tc3_body(x1max_ref, x1mean_ref, x2max_ref, x2mean_ref,
              land_ref, betti_ref,
              m1wa_ref, m1wb_ref, m1wc_ref, m1wd_ref, m1we_ref, m1wf_ref,
              g0a_ref, g0b_ref, g0c_ref, g0d_ref, g0e_ref, g0f_ref,
              b0a_ref, b0b_ref, b0c_ref, b0d_ref, b0e_ref, b0f_ref,
              m1b_ref, bn1g_ref, bn1b_ref, m2w_ref, m2b_ref,
              clw_ref, clb_ref, cls_ref):
    pieces = [
        (x1max_ref[...], m1wa_ref, g0a_ref, b0a_ref),
        (x1mean_ref[...], m1wb_ref, g0b_ref, b0b_ref),
        (x2max_ref[...], m1wc_ref, g0c_ref, b0c_ref),
        (x2mean_ref[...], m1wd_ref, g0d_ref, b0d_ref),
        (land_ref[...], m1we_ref, g0e_ref, b0e_ref),
        (betti_ref[...], m1wf_ref, g0f_ref, b0f_ref),
    ]
    f = m1b_ref[...]
    for val, wref, gref, bref in pieces:
        v = val * _BN_SCALE * gref[...] + bref[...]
        f = f + _matT(v, wref[...])
    f = jnp.maximum(f, 0.0)
    f = f * _BN_SCALE * bn1g_ref[...] + bn1b_ref[...]
    f = jnp.maximum(_matT(f, m2w_ref[...]) + m2b_ref[...], 0.0)
    logits = _matT(f, clw_ref[...]) + clb_ref[...]
    m = jnp.max(logits, axis=1, keepdims=True)
    lse = m + jnp.log(jnp.sum(jnp.exp(logits - m), axis=1, keepdims=True))
    cls_ref[...] = logits - lse


def _row2(v):
    return v.reshape(1, -1)


def kernel(x, edge_index, batch, edge_attr, PH1_feat, PH0_feat, params,
           vae_eps):
    p = params
    row = edge_index[0].reshape(NW, EPW)
    col = edge_index[1].reshape(NW, NCHUNK, K)
    w = edge_attr.reshape(NW, EPW)

    deg2 = _deg_call()(col, w)
    deg0 = deg2[:NPAD].reshape(NPAD, 1)
    deg1 = deg2[NPAD:].reshape(NPAD, 1)

    tc1 = pl.pallas_call(
        _tc1_body,
        out_shape=[
            jax.ShapeDtypeStruct((N, NHID), jnp.float32),
            jax.ShapeDtypeStruct((NPAD, 1), jnp.float32),
            jax.ShapeDtypeStruct((B, 16), jnp.float32),
            jax.ShapeDtypeStruct((B, 16), jnp.float32),
            jax.ShapeDtypeStruct((B, 1000), jnp.float32),
            jax.ShapeDtypeStruct((B, 32), jnp.float32),
        ],
    )
    hs, dinv, mu, land, ldec, betti = tc1(
        x, p['c1_W'], deg0, deg1, PH1_feat, PH0_feat, vae_eps,
        p['e0_W'], _row2(p['e0_b']), p['e1_W'], _row2(p['e1_b']),
        p['d0_W'], _row2(p['d0_b']), p['d1_W'], _row2(p['d1_b']),
        p['ct0_W'], _row2(p['ct0_b']), _row2(p['ct_bn1_g']),
        _row2(p['ct_bn1_b']), _row2(p['ct_bn2_g']), _row2(p['ct_bn2_b']),
        p['ct1_W'], _row2(p['ct1_b']))

    dinv_flat = dinv.reshape(NPAD)
    acc1 = _make_mp_call(NHID)(hs, row, col, w, dinv_flat)

    tc2 = pl.pallas_call(
        _tc2_body,
        out_shape=[
            jax.ShapeDtypeStruct((N, NHID), jnp.float32),
            jax.ShapeDtypeStruct((B, NHID), jnp.float32),
            jax.ShapeDtypeStruct((N, OUTF), jnp.float32),
        ],
    )
    batch2 = batch.reshape(N, 1)
    h1, x1mean, h1s = tc2(acc1[:N], acc1[NPAD:NPAD + N], hs, dinv[:N],
                          _row2(p['c1_b']), p['c2_W'], batch2)
    x1max = _maxpool(h1, batch2, NHID)

    acc2 = _make_mp_call(OUTF)(h1s, row, col, w, dinv_flat)

    tc3a = pl.pallas_call(
        _tc3a_body,
        out_shape=[
            jax.ShapeDtypeStruct((N, OUTF), jnp.float32),
            jax.ShapeDtypeStruct((B, OUTF), jnp.float32),
        ],
    )
    h2, x2mean = tc3a(acc2[:N], acc2[NPAD:NPAD + N], h1s, dinv[:N],
                      _row2(p['c2_b']), batch2)
    x2max = _maxpool(h2, batch2, OUTF)

    m1 = p['m1_W']
    g0 = p['m_bn0_g']
    b0 = p['m_bn0_b']
    splits = [0, 64, 128, 144, 160, 176, 208]
    m1w = [m1[:, splits[i]:splits[i + 1]] for i in range(6)]
    g0s = [_row2(g0[splits[i]:splits[i + 1]]) for i in range(6)]
    b0s = [_row2(b0[splits[i]:splits[i + 1]]) for i in range(6)]

    tc3 = pl.pallas_call(
        _tc3_body,
        out_shape=[jax.ShapeDtypeStruct((B, 2), jnp.float32)],
    )
    (cls,) = tc3(x1max, x1mean, x2max, x2mean, land, betti,
                 *m1w, *g0s, *b0s,
                 _row2(p['m1_b']), _row2(p['m_bn1_g']), _row2(p['m_bn1_b']),
                 p['m2_W'], _row2(p['m2_b']), p['cl_W'], _row2(p['cl_b']))

    return (cls, mu, mu, land, ldec, betti)


# speedup vs baseline: 25.5699x; 1.1930x over previous
import functools

import jax
import jax.numpy as jnp
from jax import lax
from jax.experimental import pallas as pl
from jax.experimental.pallas import tpu as pltpu
from jax.experimental.pallas import tpu_sc as plsc

N = 10000
NPAD = 10240
E = 320000
B = 64
INFEAT = 128
NHID = 64
OUTF = 16

NC = 2
NS = 16
NW = NC * NS
EPW = E // NW
K = 80
NCHUNK = EPW // K
RPT = NPAD // NS

_MESH = dict(core_axis_name="c", subcore_axis_name="s", num_cores=NC,
             num_subcores=NS)
_SC_PARAMS = pltpu.CompilerParams(needs_layout_passes=False,
                                  use_tc_tiling_on_sc=False)

_NEG_INF = float("-inf")
_BN_SCALE = 1.0 / (1.0 + 1e-5) ** 0.5


def _iota16():
    return lax.broadcasted_iota(jnp.int32, (16,), 0)


def _zeros16(dtype=jnp.float32):
    return jnp.zeros((16,), dtype)


def _deg_body(col_hbm, w_hbm, deg_hbm, col_v, w_v, zb, deg_s):
    cid = lax.axis_index("c")
    sid = lax.axis_index("s")
    wid = sid * NC + cid
    pltpu.sync_copy(col_hbm.at[wid], col_v)
    pltpu.sync_copy(w_hbm.at[wid], w_v)

    def zrow(r, c):
        zb[pl.ds(r * 16, 16)] = _zeros16()
        return c
    lax.fori_loop(0, RPT // 16, zrow, 0)
    pltpu.sync_copy(zb, deg_s.at[pl.ds(sid * RPT, RPT)])
    plsc.subcore_barrier()

    def chunk(i, c):
        pltpu.sync_copy(w_v.at[pl.ds(i * K, K)], deg_s.at[col_v.at[i]],
                        add=True)
        return c
    lax.fori_loop(0, NCHUNK, chunk, 0)
    plsc.subcore_barrier()
    pltpu.sync_copy(deg_s.at[pl.ds(sid * RPT, RPT)],
                    deg_hbm.at[pl.ds(cid * NPAD + sid * RPT, RPT)])


@functools.cache
def _deg_call():
    return pl.kernel(
        _deg_body,
        out_type=jax.ShapeDtypeStruct((2 * NPAD,), jnp.float32),
        mesh=plsc.VectorSubcoreMesh(**_MESH),
        compiler_params=_SC_PARAMS,
        scratch_types=[
            pltpu.VMEM((NCHUNK, K), jnp.int32),
            pltpu.VMEM((EPW,), jnp.float32),
            pltpu.VMEM((RPT,), jnp.float32),
            pltpu.VMEM_SHARED((NPAD,), jnp.float32),
        ],
    )


def _mp_body(F, hs_hbm, row_hbm, col_hbm, w_hbm, dinv_hbm, out_hbm,
             row_v, col_v, w_v, s_v, dinv_v, rb0, rb1, acc_s, sem0, sem1):
    G = F // 16
    U = 4
    cid = lax.axis_index("c")
    sid = lax.axis_index("s")
    wid = sid * NC + cid
    pltpu.sync_copy(row_hbm.at[wid], row_v)
    pltpu.sync_copy(col_hbm.at[wid], col_v)
    pltpu.sync_copy(w_hbm.at[wid], w_v)
    pltpu.sync_copy(dinv_hbm, dinv_v)

    def zrow(r, c):
        for g in range(G):
            rb0[r, pl.ds(g * 16, 16)] = _zeros16()
        return c
    lax.fori_loop(0, K, zrow, 0)
    for r8 in range(RPT // K):
        pltpu.sync_copy(rb0, acc_s.at[pl.ds(sid * RPT + r8 * K, K)])
    plsc.subcore_barrier()

    pltpu.async_copy(hs_hbm.at[row_v.at[pl.ds(0, K)]], rb0, sem0)

    def srow(i, c):
        c16 = col_v[i // (K // 16), pl.ds((i % (K // 16)) * 16, 16)]
        d16 = plsc.load_gather(dinv_v, [c16])
        s_v[pl.ds(i * 16, 16)] = w_v[pl.ds(i * 16, 16)] * d16
        return c
    lax.fori_loop(0, EPW // 16, srow, 0)

    def scale_scatter(i, buf):
        def scale(t, c2):
            e = t * U
            for u in range(U):
                bc = plsc.load_gather(
                    s_v, [jnp.full((16,), i * K + e + u, jnp.int32)])
                for g in range(G):
                    buf[e + u, pl.ds(g * 16, 16)] = (
                        buf[e + u, pl.ds(g * 16, 16)] * bc)
            return c2
        lax.fori_loop(0, K // U, scale, 0)
        pltpu.sync_copy(buf, acc_s.at[col_v.at[i]], add=True)

    def pair(g, c):
        i0 = 2 * g
        pltpu.make_async_copy(hs_hbm.at[pl.ds(0, K)], rb0, sem0).wait()
        pltpu.async_copy(hs_hbm.at[row_v.at[pl.ds((i0 + 1) * K, K)]], rb1,
                         sem1)
        scale_scatter(i0, rb0)
        pltpu.make_async_copy(hs_hbm.at[pl.ds(0, K)], rb1, sem1).wait()
        pltpu.async_copy(hs_hbm.at[row_v.at[pl.ds((i0 + 2) * K, K)]], rb0,
                         sem0)
        scale_scatter(i0 + 1, rb1)
        return c
    lax.fori_loop(0, (NCHUNK - 1) // 2, pair, 0)
    pltpu.make_async_copy(hs_hbm.at[pl.ds(0, K)], rb0, sem0).wait()
    scale_scatter(NCHUNK - 1, rb0)

    plsc.subcore_barrier()
    pltpu.sync_copy(acc_s.at[pl.ds(sid * RPT, RPT)],
                    out_hbm.at[pl.ds(cid * NPAD + sid * RPT, RPT)])


RW = NPAD // NW


def _fuse_body(F, relu, acc_hbm, hs_hbm, dinv_hbm, b_hbm, batch_hbm,
               hout_hbm, maxout_hbm,
               acc0_v, acc1_v, hs_v, dinv_v, batch_v, b_v, maxtab):
    G = F // 16
    cid = lax.axis_index("c")
    sid = lax.axis_index("s")
    wid = sid * NC + cid
    base = wid * RW
    pltpu.sync_copy(acc_hbm.at[pl.ds(base, RW)], acc0_v)
    pltpu.sync_copy(acc_hbm.at[pl.ds(NPAD + base, RW)], acc1_v)
    pltpu.sync_copy(hs_hbm.at[pl.ds(base, RW)], hs_v)
    pltpu.sync_copy(dinv_hbm.at[pl.ds(base, RW)], dinv_v)
    pltpu.sync_copy(batch_hbm.at[pl.ds(base, RW)], batch_v)
    pltpu.sync_copy(b_hbm, b_v)

    def mrow(r, c):
        for g in range(G):
            maxtab[r, pl.ds(g * 16, 16)] = jnp.full((16,), _NEG_INF,
                                                    jnp.float32)
        return c
    lax.fori_loop(0, B, mrow, 0)

    def grp(g, c):
        b16 = batch_v[pl.ds(g * 16, 16)]
        d16 = dinv_v[pl.ds(g * 16, 16)]
        for u in range(16):
            r = g * 16 + u
            bid = b16[u]
            dv = d16[u]
            bidc = jnp.maximum(bid, 0)
            valid = bid >= 0
            for g2 in range(G):
                sl = pl.ds(g2 * 16, 16)
                v = (acc0_v[r, sl] + acc1_v[r, sl] + dv * hs_v[r, sl]
                     + b_v[sl])
                if relu:
                    v = jnp.maximum(v, 0.0)
                acc0_v[r, sl] = v
                mv = jnp.where(valid, v, _NEG_INF)
                maxtab[bidc, sl] = jnp.maximum(maxtab[bidc, sl], mv)
        return c
    lax.fori_loop(0, RW // 16, grp, 0)
    pltpu.sync_copy(acc0_v, hout_hbm.at[pl.ds(base, RW)])
    pltpu.sync_copy(maxtab, maxout_hbm.at[pl.ds(wid * B, B)])


@functools.cache
def _make_fuse_call(F, relu):
    return pl.kernel(
        functools.partial(_fuse_body, F, relu),
        out_type=[
            jax.ShapeDtypeStruct((NPAD, F), jnp.float32),
            jax.ShapeDtypeStruct((NW * B, F), jnp.float32),
        ],
        mesh=plsc.VectorSubcoreMesh(**_MESH),
        compiler_params=_SC_PARAMS,
        scratch_types=[
            pltpu.VMEM((RW, F), jnp.float32),
            pltpu.VMEM((RW, F), jnp.float32),
            pltpu.VMEM((RW, F), jnp.float32),
            pltpu.VMEM((RW,), jnp.float32),
            pltpu.VMEM((RW,), jnp.int32),
            pltpu.VMEM((F,), jnp.float32),
            pltpu.VMEM((B, F), jnp.float32),
        ],
    )


@functools.cache
def _make_mp_call(F):
    return pl.kernel(
        functools.partial(_mp_body, F),
        out_type=jax.ShapeDtypeStruct((2 * NPAD, F), jnp.float32),
        mesh=plsc.VectorSubcoreMesh(**_MESH),
        compiler_params=_SC_PARAMS,
        scratch_types=[
            pltpu.VMEM((EPW,), jnp.int32),
            pltpu.VMEM((NCHUNK, K), jnp.int32),
            pltpu.VMEM((EPW,), jnp.float32),
            pltpu.VMEM((EPW,), jnp.float32),
            pltpu.VMEM((NPAD,), jnp.float32),
            pltpu.VMEM((K, F), jnp.float32),
            pltpu.VMEM((K, F), jnp.float32),
            pltpu.VMEM_SHARED((NPAD, F), jnp.float32),
            pltpu.SemaphoreType.DMA,
            pltpu.SemaphoreType.DMA,
        ],
    )


def _matT(a, w):
    return lax.dot_general(a, w, (((1,), (1,)), ((), ())),
                           preferred_element_type=jnp.float32)


def _lrelu(x):
    return jnp.where(x > 0, x, 0.01 * x)


def _tc1_body(x_ref, w1_ref, deg0_ref, deg1_ref, hs_ref, dinv_ref):
    deg = 1.0 + deg0_ref[...] + deg1_ref[...]
    dinv = lax.rsqrt(deg)
    dinv_ref[...] = dinv
    h0 = _matT(x_ref[...], w1_ref[...])
    hs_ref[pl.ds(0, N), :] = h0 * dinv[:N]
    hs_ref[pl.ds(N, NPAD - N), :] = jnp.zeros((NPAD - N, NHID), jnp.float32)


def _dense_body(ph1_ref, ph0_ref, eps_ref,
                e0w_ref, e0b_ref, e1w_ref, e1b_ref, d0w_ref, d0b_ref,
                d1w_ref, d1b_ref, ct0w_ref, ct0b_ref, bn1g_ref, bn1b_ref,
                bn2g_ref, bn2b_ref, ct1w_ref, ct1b_ref,
                mu_ref, land_ref, ldec_ref, betti_ref):
    emb = _lrelu(_matT(ph1_ref[...], e0w_ref[...]) + e0b_ref[...])
    mu = _matT(emb, e1w_ref[...]) + e1b_ref[...]
    mu_ref[...] = mu
    std = jnp.exp(0.5 * mu)
    land = eps_ref[...] * std + mu
    land_ref[...] = land
    dec = _lrelu(_matT(land, d0w_ref[...]) + d0b_ref[...])
    ld = _matT(dec, d1w_ref[...]) + d1b_ref[...]
    ldec_ref[...] = 1.0 / (1.0 + jnp.exp(-ld))

    bc = _matT(ph0_ref[...], ct0w_ref[...]) + ct0b_ref[...]
    bc = bc * _BN_SCALE * bn1g_ref[...] + bn1b_ref[...]
    bc = _lrelu(bc)
    bc = bc * _BN_SCALE * bn2g_ref[...] + bn2b_ref[...]
    betti_ref[...] = _matT(bc, ct1w_ref[...]) + ct1b_ref[...]


def _seg_mean(h, batch):
    maskf = (batch == lax.broadcasted_iota(jnp.int32, (N, B), 1)
             ).astype(jnp.float32)
    sums = lax.dot_general(maskf, h, (((0,), (0,)), ((), ())),
                           preferred_element_type=jnp.float32)
    cnt = lax.dot_general(maskf, jnp.ones((N, 1), jnp.float32),
                          (((0,), (0,)), ((), ())),
                          preferred_element_type=jnp.float32)
    return sums / jnp.maximum(cnt, 1.0)


def _maxcomb(parts_ref):
    mx = parts_ref[pl.ds(0, B), :]
    for t in range(1, NW):
        mx = jnp.maximum(mx, parts_ref[pl.ds(t * B, B), :])
    return mx


def _tc2_body(h1_ref, dinv_ref, w2_ref, batch_ref, max1_ref,
              x1max_ref, x1mean_ref, h1s_ref):
    h1 = h1_ref[...]
    x1max_ref[...] = _maxcomb(max1_ref)
    x1mean_ref[...] = _seg_mean(h1[:N], batch_ref[...])
    h1s_ref[...] = _matT(h1, w2_ref[...]) * dinv_ref[...]


def _tc3_body(h2_ref, batch_ref, max2_ref, x1max_ref, x1mean_ref,
              land_ref, betti_ref,
              m1wa_ref, m1wb_ref, m1wc_ref, m1wd_ref, m1we_ref, m1wf_ref,
              g0a_ref, g0b_ref, g0c_ref, g0d_ref, g0e_ref, g0f_ref,
              b0a_ref, b0b_ref, b0c_ref, b0d_ref, b0e_ref, b0f_ref,
              m1b_ref, bn1g_ref, bn1b_ref, m2w_ref, m2b_ref,
              clw_ref, clb_ref, cls_ref):
    h2 = h2_ref[...]
    x2max = _maxcomb(max2_ref)
    x2mean = _seg_mean(h2[:N], batch_ref[...])
    pieces = [
        (x1max_ref[...], m1wa_ref, g0a_ref, b0a_ref),
        (x1mean_ref[...], m1wb_ref, g0b_ref, b0b_ref),
        (x2max, m1wc_ref, g0c_ref, b0c_ref),
        (x2mean, m1wd_ref, g0d_ref, b0d_ref),
        (land_ref[...], m1we_ref, g0e_ref, b0e_ref),
        (betti_ref[...], m1wf_ref, g0f_ref, b0f_ref),
    ]
    f = m1b_ref[...]
    for val, wref, gref, bref in pieces:
        v = val * _BN_SCALE * gref[...] + bref[...]
        f = f + _matT(v, wref[...])
    f = jnp.maximum(f, 0.0)
    f = f * _BN_SCALE * bn1g_ref[...] + bn1b_ref[...]
    f = jnp.maximum(_matT(f, m2w_ref[...]) + m2b_ref[...], 0.0)
    logits = _matT(f, clw_ref[...]) + clb_ref[...]
    m = jnp.max(logits, axis=1, keepdims=True)
    lse = m + jnp.log(jnp.sum(jnp.exp(logits - m), axis=1, keepdims=True))
    cls_ref[...] = logits - lse


def _row2(v):
    return v.reshape(1, -1)


def kernel(x, edge_index, batch, edge_attr, PH1_feat, PH0_feat, params,
           vae_eps):
    p = params
    row = edge_index[0].reshape(NW, EPW)
    col = edge_index[1].reshape(NW, NCHUNK, K)
    w = edge_attr.reshape(NW, EPW)

    deg2 = _deg_call()(col, w)
    deg0 = deg2[:NPAD].reshape(NPAD, 1)
    deg1 = deg2[NPAD:].reshape(NPAD, 1)

    tc1 = pl.pallas_call(
        _tc1_body,
        out_shape=[
            jax.ShapeDtypeStruct((NPAD, NHID), jnp.float32),
            jax.ShapeDtypeStruct((NPAD, 1), jnp.float32),
        ],
    )
    hs, dinv = tc1(x, p['c1_W'], deg0, deg1)

    dense = pl.pallas_call(
        _dense_body,
        out_shape=[
            jax.ShapeDtypeStruct((B, 16), jnp.float32),
            jax.ShapeDtypeStruct((B, 16), jnp.float32),
            jax.ShapeDtypeStruct((B, 1000), jnp.float32),
            jax.ShapeDtypeStruct((B, 32), jnp.float32),
        ],
    )
    mu, land, ldec, betti = dense(
        PH1_feat, PH0_feat, vae_eps,
        p['e0_W'], _row2(p['e0_b']), p['e1_W'], _row2(p['e1_b']),
        p['d0_W'], _row2(p['d0_b']), p['d1_W'], _row2(p['d1_b']),
        p['ct0_W'], _row2(p['ct0_b']), _row2(p['ct_bn1_g']),
        _row2(p['ct_bn1_b']), _row2(p['ct_bn2_g']), _row2(p['ct_bn2_b']),
        p['ct1_W'], _row2(p['ct1_b']))

    dinv_flat = dinv.reshape(NPAD)
    batchp = jnp.concatenate(
        [batch, jnp.full((NPAD - N,), -1, jnp.int32)])
    batch2 = batch.reshape(N, 1)

    acc1 = _make_mp_call(NHID)(hs, row, col, w, dinv_flat)
    h1, max1 = _make_fuse_call(NHID, True)(
        acc1, hs, dinv_flat, p['c1_b'], batchp)

    tc2 = pl.pallas_call(
        _tc2_body,
        out_shape=[
            jax.ShapeDtypeStruct((B, NHID), jnp.float32),
            jax.ShapeDtypeStruct((B, NHID), jnp.float32),
            jax.ShapeDtypeStruct((NPAD, OUTF), jnp.float32),
        ],
    )
    x1max, x1mean, h1s = tc2(h1, dinv, p['c2_W'], batch2, max1)

    acc2 = _make_mp_call(OUTF)(h1s, row, col, w, dinv_flat)
    h2, max2 = _make_fuse_call(OUTF, False)(
        acc2, h1s, dinv_flat, p['c2_b'], batchp)

    m1 = p['m1_W']
    g0 = p['m_bn0_g']
    b0 = p['m_bn0_b']
    splits = [0, 64, 128, 144, 160, 176, 208]
    m1w = [m1[:, splits[i]:splits[i + 1]] for i in range(6)]
    g0s = [_row2(g0[splits[i]:splits[i + 1]]) for i in range(6)]
    b0s = [_row2(b0[splits[i]:splits[i + 1]]) for i in range(6)]

    tc3 = pl.pallas_call(
        _tc3_body,
        out_shape=[jax.ShapeDtypeStruct((B, 2), jnp.float32)],
    )
    (cls,) = tc3(h2, batch2, max2, x1max, x1mean, land, betti,
                 *m1w, *g0s, *b0s,
                 _row2(p['m1_b']), _row2(p['m_bn1_g']), _row2(p['m_bn1_b']),
                 p['m2_W'], _row2(p['m2_b']), p['cl_W'], _row2(p['cl_b']))

    return (cls, mu, mu, land, ldec, betti)


# speedup vs baseline: 27.1367x; 1.0613x over previous
import functools

import jax
import jax.numpy as jnp
from jax import lax
from jax.experimental import pallas as pl
from jax.experimental.pallas import tpu as pltpu
from jax.experimental.pallas import tpu_sc as plsc

N = 10000
NPAD = 10240
E = 320000
B = 64
INFEAT = 128
NHID = 64
OUTF = 16

NC = 2
NS = 16
NW = NC * NS
EPW = E // NW
K = 80
NCHUNK = EPW // K
RPT = NPAD // NS

_MESH = dict(core_axis_name="c", subcore_axis_name="s", num_cores=NC,
             num_subcores=NS)
_SC_PARAMS = pltpu.CompilerParams(needs_layout_passes=False,
                                  use_tc_tiling_on_sc=False)

_NEG_INF = float("-inf")
_BN_SCALE = 1.0 / (1.0 + 1e-5) ** 0.5


def _iota16():
    return lax.broadcasted_iota(jnp.int32, (16,), 0)


def _zeros16(dtype=jnp.float32):
    return jnp.zeros((16,), dtype)


def _deg_body(col_hbm, w_hbm, deg_hbm, col_v, w_v, zb, deg_s):
    cid = lax.axis_index("c")
    sid = lax.axis_index("s")
    wid = sid * NC + cid
    pltpu.sync_copy(col_hbm.at[wid], col_v)
    pltpu.sync_copy(w_hbm.at[wid], w_v)

    def zrow(r, c):
        zb[pl.ds(r * 16, 16)] = _zeros16()
        return c
    lax.fori_loop(0, RPT // 16, zrow, 0)
    pltpu.sync_copy(zb, deg_s.at[pl.ds(sid * RPT, RPT)])
    plsc.subcore_barrier()

    def chunk(i, c):
        pltpu.sync_copy(w_v.at[pl.ds(i * K, K)], deg_s.at[col_v.at[i]],
                        add=True)
        return c
    lax.fori_loop(0, NCHUNK, chunk, 0)
    plsc.subcore_barrier()
    pltpu.sync_copy(deg_s.at[pl.ds(sid * RPT, RPT)],
                    deg_hbm.at[pl.ds(cid * NPAD + sid * RPT, RPT)])


@functools.cache
def _deg_call():
    return pl.kernel(
        _deg_body,
        out_type=jax.ShapeDtypeStruct((2 * NPAD,), jnp.float32),
        mesh=plsc.VectorSubcoreMesh(**_MESH),
        compiler_params=_SC_PARAMS,
        scratch_types=[
            pltpu.VMEM((NCHUNK, K), jnp.int32),
            pltpu.VMEM((EPW,), jnp.float32),
            pltpu.VMEM((RPT,), jnp.float32),
            pltpu.VMEM_SHARED((NPAD,), jnp.float32),
        ],
    )


def _mp_body(F, hs_hbm, row_hbm, col_hbm, w_hbm, dinv_hbm, out_hbm,
             row_v, col_v, w_v, s_v, dinv_v, rb0, rb1, rb2, acc_s,
             g0sem, g1sem, g2sem, s0sem, s1sem, s2sem):
    G = F // 16
    U = 4
    cid = lax.axis_index("c")
    sid = lax.axis_index("s")
    wid = sid * NC + cid
    bufs = ((rb0, g0sem, s0sem), (rb1, g1sem, s1sem), (rb2, g2sem, s2sem))
    pltpu.sync_copy(row_hbm.at[wid], row_v)
    pltpu.sync_copy(col_hbm.at[wid], col_v)
    pltpu.sync_copy(w_hbm.at[wid], w_v)
    pltpu.sync_copy(dinv_hbm, dinv_v)

    def zrow(r, c):
        for g in range(G):
            rb0[r, pl.ds(g * 16, 16)] = _zeros16()
        return c
    lax.fori_loop(0, K, zrow, 0)
    for r8 in range(RPT // K):
        pltpu.sync_copy(rb0, acc_s.at[pl.ds(sid * RPT + r8 * K, K)])
    plsc.subcore_barrier()

    def start_gather(i, rb, gs):
        pltpu.async_copy(hs_hbm.at[row_v.at[pl.ds(i * K, K)]], rb, gs)

    start_gather(0, rb0, g0sem)
    start_gather(1, rb1, g1sem)

    def srow(i, c):
        c16 = col_v[i // (K // 16), pl.ds((i % (K // 16)) * 16, 16)]
        d16 = plsc.load_gather(dinv_v, [c16])
        s_v[pl.ds(i * 16, 16)] = w_v[pl.ds(i * 16, 16)] * d16
        return c
    lax.fori_loop(0, EPW // 16, srow, 0)

    def scale(i, buf):
        def body(t, c2):
            e = t * U
            for u in range(U):
                bc = plsc.load_gather(
                    s_v, [jnp.full((16,), i * K + e + u, jnp.int32)])
                for g in range(G):
                    buf[e + u, pl.ds(g * 16, 16)] = (
                        buf[e + u, pl.ds(g * 16, 16)] * bc)
            return c2
        lax.fori_loop(0, K // U, body, 0)

    def do_chunk(i, u, scwait):
        rb, gs, ss = bufs[u]
        v = (u + 2) % 3
        rbv, gsv, ssv = bufs[v]
        pltpu.make_async_copy(hs_hbm.at[pl.ds(0, K)], rb, gs).wait()
        if scwait:
            pltpu.make_async_copy(rbv, acc_s.at[pl.ds(0, K)], ssv).wait()

        more = i + 2 < NCHUNK
        if isinstance(more, bool):
            if more:
                start_gather(i + 2, rbv, gsv)
        else:
            @pl.when(more)
            def _():
                start_gather(i + 2, rbv, gsv)
        scale(i, rb)
        pltpu.async_copy(rb, acc_s.at[col_v.at[i]], ss, add=True)

    do_chunk(0, 0, False)

    def tri(t, c):
        i0 = 3 * t + 1
        do_chunk(i0, 1, True)
        do_chunk(i0 + 1, 2, True)
        do_chunk(i0 + 2, 0, True)
        return c
    lax.fori_loop(0, (NCHUNK - 2) // 3, tri, 0)
    do_chunk(NCHUNK - 1, (NCHUNK - 1) % 3, True)

    lrb, _lgs, lss = bufs[(NCHUNK - 1) % 3]
    pltpu.make_async_copy(lrb, acc_s.at[pl.ds(0, K)], lss).wait()

    plsc.subcore_barrier()
    pltpu.sync_copy(acc_s.at[pl.ds(sid * RPT, RPT)],
                    out_hbm.at[pl.ds(cid * NPAD + sid * RPT, RPT)])


RW = NPAD // NW


def _fuse_body(F, relu, acc_hbm, hs_hbm, dinv_hbm, b_hbm, batch_hbm,
               hout_hbm, maxout_hbm,
               acc0_v, acc1_v, hs_v, dinv_v, batch_v, b_v, maxtab):
    G = F // 16
    cid = lax.axis_index("c")
    sid = lax.axis_index("s")
    wid = sid * NC + cid
    base = wid * RW
    pltpu.sync_copy(acc_hbm.at[pl.ds(base, RW)], acc0_v)
    pltpu.sync_copy(acc_hbm.at[pl.ds(NPAD + base, RW)], acc1_v)
    pltpu.sync_copy(hs_hbm.at[pl.ds(base, RW)], hs_v)
    pltpu.sync_copy(dinv_hbm.at[pl.ds(base, RW)], dinv_v)
    pltpu.sync_copy(batch_hbm.at[pl.ds(base, RW)], batch_v)
    pltpu.sync_copy(b_hbm, b_v)

    def mrow(r, c):
        for g in range(G):
            maxtab[r, pl.ds(g * 16, 16)] = jnp.full((16,), _NEG_INF,
                                                    jnp.float32)
        return c
    lax.fori_loop(0, B, mrow, 0)

    def grp(g, c):
        b16 = batch_v[pl.ds(g * 16, 16)]
        d16 = dinv_v[pl.ds(g * 16, 16)]
        for u in range(16):
            r = g * 16 + u
            bid = b16[u]
            dv = d16[u]
            bidc = jnp.maximum(bid, 0)
            valid = bid >= 0
            for g2 in range(G):
                sl = pl.ds(g2 * 16, 16)
                v = (acc0_v[r, sl] + acc1_v[r, sl] + dv * hs_v[r, sl]
                     + b_v[sl])
                if relu:
                    v = jnp.maximum(v, 0.0)
                acc0_v[r, sl] = v
                mv = jnp.where(valid, v, _NEG_INF)
                maxtab[bidc, sl] = jnp.maximum(maxtab[bidc, sl], mv)
        return c
    lax.fori_loop(0, RW // 16, grp, 0)
    pltpu.sync_copy(acc0_v, hout_hbm.at[pl.ds(base, RW)])
    pltpu.sync_copy(maxtab, maxout_hbm.at[pl.ds(wid * B, B)])


@functools.cache
def _make_fuse_call(F, relu):
    return pl.kernel(
        functools.partial(_fuse_body, F, relu),
        out_type=[
            jax.ShapeDtypeStruct((NPAD, F), jnp.float32),
            jax.ShapeDtypeStruct((NW * B, F), jnp.float32),
        ],
        mesh=plsc.VectorSubcoreMesh(**_MESH),
        compiler_params=_SC_PARAMS,
        scratch_types=[
            pltpu.VMEM((RW, F), jnp.float32),
            pltpu.VMEM((RW, F), jnp.float32),
            pltpu.VMEM((RW, F), jnp.float32),
            pltpu.VMEM((RW,), jnp.float32),
            pltpu.VMEM((RW,), jnp.int32),
            pltpu.VMEM((F,), jnp.float32),
            pltpu.VMEM((B, F), jnp.float32),
        ],
    )


@functools.cache
def _make_mp_call(F):
    return pl.kernel(
        functools.partial(_mp_body, F),
        out_type=jax.ShapeDtypeStruct((2 * NPAD, F), jnp.float32),
        mesh=plsc.VectorSubcoreMesh(**_MESH),
        compiler_params=_SC_PARAMS,
        scratch_types=[
            pltpu.VMEM((EPW,), jnp.int32),
            pltpu.VMEM((NCHUNK, K), jnp.int32),
            pltpu.VMEM((EPW,), jnp.float32),
            pltpu.VMEM((EPW,), jnp.float32),
            pltpu.VMEM((NPAD,), jnp.float32),
            pltpu.VMEM((K, F), jnp.float32),
            pltpu.VMEM((K, F), jnp.float32),
            pltpu.VMEM((K, F), jnp.float32),
            pltpu.VMEM_SHARED((NPAD, F), jnp.float32),
            pltpu.SemaphoreType.DMA,
            pltpu.SemaphoreType.DMA,
            pltpu.SemaphoreType.DMA,
            pltpu.SemaphoreType.DMA,
            pltpu.SemaphoreType.DMA,
            pltpu.SemaphoreType.DMA,
        ],
    )


def _matT(a, w):
    return lax.dot_general(a, w, (((1,), (1,)), ((), ())),
                           preferred_element_type=jnp.float32)


def _lrelu(x):
    return jnp.where(x > 0, x, 0.01 * x)


def _tc1_body(x_ref, w1_ref, deg0_ref, deg1_ref, hs_ref, dinv_ref):
    deg = 1.0 + deg0_ref[...] + deg1_ref[...]
    dinv = lax.rsqrt(deg)
    dinv_ref[...] = dinv
    h0 = _matT(x_ref[...], w1_ref[...])
    hs_ref[pl.ds(0, N), :] = h0 * dinv[:N]
    hs_ref[pl.ds(N, NPAD - N), :] = jnp.zeros((NPAD - N, NHID), jnp.float32)


def _dense_body(ph1_ref, ph0_ref, eps_ref,
                e0w_ref, e0b_ref, e1w_ref, e1b_ref, d0w_ref, d0b_ref,
                d1w_ref, d1b_ref, ct0w_ref, ct0b_ref, bn1g_ref, bn1b_ref,
                bn2g_ref, bn2b_ref, ct1w_ref, ct1b_ref,
                mu_ref, land_ref, ldec_ref, betti_ref):
    emb = _lrelu(_matT(ph1_ref[...], e0w_ref[...]) + e0b_ref[...])
    mu = _matT(emb, e1w_ref[...]) + e1b_ref[...]
    mu_ref[...] = mu
    std = jnp.exp(0.5 * mu)
    land = eps_ref[...] * std + mu
    land_ref[...] = land
    dec = _lrelu(_matT(land, d0w_ref[...]) + d0b_ref[...])
    ld = _matT(dec, d1w_ref[...]) + d1b_ref[...]
    ldec_ref[...] = 1.0 / (1.0 + jnp.exp(-ld))

    bc = _matT(ph0_ref[...], ct0w_ref[...]) + ct0b_ref[...]
    bc = bc * _BN_SCALE * bn1g_ref[...] + bn1b_ref[...]
    bc = _lrelu(bc)
    bc = bc * _BN_SCALE * bn2g_ref[...] + bn2b_ref[...]
    betti_ref[...] = _matT(bc, ct1w_ref[...]) + ct1b_ref[...]


def _seg_mean(h, batch):
    maskf = (batch == lax.broadcasted_iota(jnp.int32, (N, B), 1)
             ).astype(jnp.float32)
    sums = lax.dot_general(maskf, h, (((0,), (0,)), ((), ())),
                           preferred_element_type=jnp.float32)
    cnt = lax.dot_general(maskf, jnp.ones((N, 1), jnp.float32),
                          (((0,), (0,)), ((), ())),
                          preferred_element_type=jnp.float32)
    return sums / jnp.maximum(cnt, 1.0)


def _maxcomb(parts_ref):
    mx = parts_ref[pl.ds(0, B), :]
    for t in range(1, NW):
        mx = jnp.maximum(mx, parts_ref[pl.ds(t * B, B), :])
    return mx


def _tc2_body(h1_ref, dinv_ref, w2_ref, batch_ref, max1_ref,
              x1max_ref, x1mean_ref, h1s_ref):
    h1 = h1_ref[...]
    x1max_ref[...] = _maxcomb(max1_ref)
    x1mean_ref[...] = _seg_mean(h1[:N], batch_ref[...])
    h1s_ref[...] = _matT(h1, w2_ref[...]) * dinv_ref[...]


def _tc3_body(h2_ref, batch_ref, max2_ref, x1max_ref, x1mean_ref,
              land_ref, betti_ref,
              m1wa_ref, m1wb_ref, m1wc_ref, m1wd_ref, m1we_ref, m1wf_ref,
              g0a_ref, g0b_ref, g0c_ref, g0d_ref, g0e_ref, g0f_ref,
              b0a_ref, b0b_ref, b0c_ref, b0d_ref, b0e_ref, b0f_ref,
              m1b_ref, bn1g_ref, bn1b_ref, m2w_ref, m2b_ref,
              clw_ref, clb_ref, cls_ref):
    h2 = h2_ref[...]
    x2max = _maxcomb(max2_ref)
    x2mean = _seg_mean(h2[:N], batch_ref[...])
    pieces = [
        (x1max_ref[...], m1wa_ref, g0a_ref, b0a_ref),
        (x1mean_ref[...], m1wb_ref, g0b_ref, b0b_ref),
        (x2max, m1wc_ref, g0c_ref, b0c_ref),
        (x2mean, m1wd_ref, g0d_ref, b0d_ref),
        (land_ref[...], m1we_ref, g0e_ref, b0e_ref),
        (betti_ref[...], m1wf_ref, g0f_ref, b0f_ref),
    ]
    f = m1b_ref[...]
    for val, wref, gref, bref in pieces:
        v = val * _BN_SCALE * gref[...] + bref[...]
        f = f + _matT(v, wref[...])
    f = jnp.maximum(f, 0.0)
    f = f * _BN_SCALE * bn1g_ref[...] + bn1b_ref[...]
    f = jnp.maximum(_matT(f, m2w_ref[...]) + m2b_ref[...], 0.0)
    logits = _matT(f, clw_ref[...]) + clb_ref[...]
    m = jnp.max(logits, axis=1, keepdims=True)
    lse = m + jnp.log(jnp.sum(jnp.exp(logits - m), axis=1, keepdims=True))
    cls_ref[...] = logits - lse


def _row2(v):
    return v.reshape(1, -1)


def kernel(x, edge_index, batch, edge_attr, PH1_feat, PH0_feat, params,
           vae_eps):
    p = params
    row = edge_index[0].reshape(NW, EPW)
    col = edge_index[1].reshape(NW, NCHUNK, K)
    w = edge_attr.reshape(NW, EPW)

    deg2 = _deg_call()(col, w)
    deg0 = deg2[:NPAD].reshape(NPAD, 1)
    deg1 = deg2[NPAD:].reshape(NPAD, 1)

    tc1 = pl.pallas_call(
        _tc1_body,
        out_shape=[
            jax.ShapeDtypeStruct((NPAD, NHID), jnp.float32),
            jax.ShapeDtypeStruct((NPAD, 1), jnp.float32),
        ],
    )
    hs, dinv = tc1(x, p['c1_W'], deg0, deg1)

    dense = pl.pallas_call(
        _dense_body,
        out_shape=[
            jax.ShapeDtypeStruct((B, 16), jnp.float32),
            jax.ShapeDtypeStruct((B, 16), jnp.float32),
            jax.ShapeDtypeStruct((B, 1000), jnp.float32),
            jax.ShapeDtypeStruct((B, 32), jnp.float32),
        ],
    )
    mu, land, ldec, betti = dense(
        PH1_feat, PH0_feat, vae_eps,
        p['e0_W'], _row2(p['e0_b']), p['e1_W'], _row2(p['e1_b']),
        p['d0_W'], _row2(p['d0_b']), p['d1_W'], _row2(p['d1_b']),
        p['ct0_W'], _row2(p['ct0_b']), _row2(p['ct_bn1_g']),
        _row2(p['ct_bn1_b']), _row2(p['ct_bn2_g']), _row2(p['ct_bn2_b']),
        p['ct1_W'], _row2(p['ct1_b']))

    dinv_flat = dinv.reshape(NPAD)
    batchp = jnp.concatenate(
        [batch, jnp.full((NPAD - N,), -1, jnp.int32)])
    batch2 = batch.reshape(N, 1)

    acc1 = _make_mp_call(NHID)(hs, row, col, w, dinv_flat)
    h1, max1 = _make_fuse_call(NHID, True)(
        acc1, hs, dinv_flat, p['c1_b'], batchp)

    tc2 = pl.pallas_call(
        _tc2_body,
        out_shape=[
            jax.ShapeDtypeStruct((B, NHID), jnp.float32),
            jax.ShapeDtypeStruct((B, NHID), jnp.float32),
            jax.ShapeDtypeStruct((NPAD, OUTF), jnp.float32),
        ],
    )
    x1max, x1mean, h1s = tc2(h1, dinv, p['c2_W'], batch2, max1)

    acc2 = _make_mp_call(OUTF)(h1s, row, col, w, dinv_flat)
    h2, max2 = _make_fuse_call(OUTF, False)(
        acc2, h1s, dinv_flat, p['c2_b'], batchp)

    m1 = p['m1_W']
    g0 = p['m_bn0_g']
    b0 = p['m_bn0_b']
    splits = [0, 64, 128, 144, 160, 176, 208]
    m1w = [m1[:, splits[i]:splits[i + 1]] for i in range(6)]
    g0s = [_row2(g0[splits[i]:splits[i + 1]]) for i in range(6)]
    b0s = [_row2(b0[splits[i]:splits[i + 1]]) for i in range(6)]

    tc3 = pl.pallas_call(
        _tc3_body,
        out_shape=[jax.ShapeDtypeStruct((B, 2), jnp.float32)],
    )
    (cls,) = tc3(h2, batch2, max2, x1max, x1mean, land, betti,
                 *m1w, *g0s, *b0s,
                 _row2(p['m1_b']), _row2(p['m_bn1_g']), _row2(p['m_bn1_b']),
                 p['m2_W'], _row2(p['m2_b']), p['cl_W'], _row2(p['cl_b']))

    return (cls, mu, mu, land, ldec, betti)


# speedup vs baseline: 29.0479x; 1.0704x over previous
import functools

import jax
import jax.numpy as jnp
from jax import lax
from jax.experimental import pallas as pl
from jax.experimental.pallas import tpu as pltpu
from jax.experimental.pallas import tpu_sc as plsc

N = 10000
NPAD = 10240
E = 320000
B = 64
INFEAT = 128
NHID = 64
OUTF = 16

NC = 2
NS = 16
NW = NC * NS
EPW = E // NW
K = 80
NCHUNK = EPW // K
RPT = NPAD // NS

_MESH = dict(core_axis_name="c", subcore_axis_name="s", num_cores=NC,
             num_subcores=NS)
_SC_PARAMS = pltpu.CompilerParams(needs_layout_passes=False,
                                  use_tc_tiling_on_sc=False)

_NEG_INF = float("-inf")
_BN_SCALE = 1.0 / (1.0 + 1e-5) ** 0.5


def _iota16():
    return lax.broadcasted_iota(jnp.int32, (16,), 0)


def _zeros16(dtype=jnp.float32):
    return jnp.zeros((16,), dtype)


def _rsqrt16(x):
    i = plsc.bitcast(x, jnp.int32)
    y = plsc.bitcast(jnp.int32(0x5F3759DF) - lax.shift_right_arithmetic(i, 1),
                     jnp.float32)
    for _ in range(3):
        y = y * (1.5 - 0.5 * x * y * y)
    return y


def _dinv16(p0, p1):
    return _rsqrt16(1.0 + p0 + p1)


def _deg_body(col_hbm, w_hbm, deg_hbm, col_v, w_v, zb, deg_s):
    cid = lax.axis_index("c")
    sid = lax.axis_index("s")
    wid = sid * NC + cid
    pltpu.sync_copy(col_hbm.at[wid], col_v)
    pltpu.sync_copy(w_hbm.at[wid], w_v)

    def zrow(r, c):
        zb[pl.ds(r * 16, 16)] = _zeros16()
        return c
    lax.fori_loop(0, RPT // 16, zrow, 0)
    pltpu.sync_copy(zb, deg_s.at[pl.ds(sid * RPT, RPT)])
    plsc.subcore_barrier()

    def chunk(i, c):
        pltpu.sync_copy(w_v.at[pl.ds(i * K, K)], deg_s.at[col_v.at[i]],
                        add=True)
        return c
    lax.fori_loop(0, NCHUNK, chunk, 0)
    plsc.subcore_barrier()
    pltpu.sync_copy(deg_s.at[pl.ds(sid * RPT, RPT)],
                    deg_hbm.at[pl.ds(cid * NPAD + sid * RPT, RPT)])


@functools.cache
def _deg_call():
    return pl.kernel(
        _deg_body,
        out_type=jax.ShapeDtypeStruct((2 * NPAD,), jnp.float32),
        mesh=plsc.VectorSubcoreMesh(**_MESH),
        compiler_params=_SC_PARAMS,
        scratch_types=[
            pltpu.VMEM((NCHUNK, K), jnp.int32),
            pltpu.VMEM((EPW,), jnp.float32),
            pltpu.VMEM((RPT,), jnp.float32),
            pltpu.VMEM_SHARED((NPAD,), jnp.float32),
        ],
    )


def _mp_body(F, hs_hbm, row_hbm, col_hbm, w_hbm, deg_hbm, out_hbm,
             row_v, col_v, w_v, s_v, dinv_v, dp0, dp1, rb0, rb1, rb2,
             acc_s, dinv_s, g0sem, g1sem, g2sem, s0sem, s1sem, s2sem):
    G = F // 16
    U = 4
    cid = lax.axis_index("c")
    sid = lax.axis_index("s")
    wid = sid * NC + cid
    bufs = ((rb0, g0sem, s0sem), (rb1, g1sem, s1sem), (rb2, g2sem, s2sem))
    pltpu.sync_copy(row_hbm.at[wid], row_v)
    pltpu.sync_copy(col_hbm.at[wid], col_v)
    pltpu.sync_copy(w_hbm.at[wid], w_v)
    pltpu.sync_copy(deg_hbm.at[pl.ds(sid * RPT, RPT)], dp0)
    pltpu.sync_copy(deg_hbm.at[pl.ds(NPAD + sid * RPT, RPT)], dp1)

    def drow(r, c):
        sl = pl.ds(r * 16, 16)
        dp0[sl] = _dinv16(dp0[sl], dp1[sl])
        return c
    lax.fori_loop(0, RPT // 16, drow, 0)
    pltpu.sync_copy(dp0, dinv_s.at[pl.ds(sid * RPT, RPT)])

    def zrow(r, c):
        for g in range(G):
            rb0[r, pl.ds(g * 16, 16)] = _zeros16()
        return c
    lax.fori_loop(0, K, zrow, 0)
    for r8 in range(RPT // K):
        pltpu.sync_copy(rb0, acc_s.at[pl.ds(sid * RPT + r8 * K, K)])
    plsc.subcore_barrier()
    pltpu.sync_copy(dinv_s, dinv_v)

    def start_gather(i, rb, gs):
        pltpu.async_copy(hs_hbm.at[row_v.at[pl.ds(i * K, K)]], rb, gs)

    start_gather(0, rb0, g0sem)
    start_gather(1, rb1, g1sem)

    def srow(i, c):
        sl = pl.ds(i * 16, 16)
        c16 = col_v[i // (K // 16), pl.ds((i % (K // 16)) * 16, 16)]
        dcol = plsc.load_gather(dinv_v, [c16])
        drow_ = plsc.load_gather(dinv_v, [row_v[sl]])
        s_v[sl] = w_v[sl] * drow_ * dcol
        return c
    lax.fori_loop(0, EPW // 16, srow, 0)

    def scale(i, buf):
        def body(t, c2):
            e = t * U
            for u in range(U):
                bc = plsc.load_gather(
                    s_v, [jnp.full((16,), i * K + e + u, jnp.int32)])
                for g in range(G):
                    buf[e + u, pl.ds(g * 16, 16)] = (
                        buf[e + u, pl.ds(g * 16, 16)] * bc)
            return c2
        lax.fori_loop(0, K // U, body, 0)

    def do_chunk(i, u, scwait):
        rb, gs, ss = bufs[u]
        v = (u + 2) % 3
        rbv, gsv, ssv = bufs[v]
        pltpu.make_async_copy(hs_hbm.at[pl.ds(0, K)], rb, gs).wait()
        if scwait:
            pltpu.make_async_copy(rbv, acc_s.at[pl.ds(0, K)], ssv).wait()

        more = i + 2 < NCHUNK
        if isinstance(more, bool):
            if more:
                start_gather(i + 2, rbv, gsv)
        else:
            @pl.when(more)
            def _():
                start_gather(i + 2, rbv, gsv)
        scale(i, rb)
        pltpu.async_copy(rb, acc_s.at[col_v.at[i]], ss, add=True)

    do_chunk(0, 0, False)

    def tri(t, c):
        i0 = 3 * t + 1
        do_chunk(i0, 1, True)
        do_chunk(i0 + 1, 2, True)
        do_chunk(i0 + 2, 0, True)
        return c
    lax.fori_loop(0, (NCHUNK - 2) // 3, tri, 0)
    do_chunk(NCHUNK - 1, (NCHUNK - 1) % 3, True)

    lrb, _lgs, lss = bufs[(NCHUNK - 1) % 3]
    pltpu.make_async_copy(lrb, acc_s.at[pl.ds(0, K)], lss).wait()

    plsc.subcore_barrier()
    pltpu.sync_copy(acc_s.at[pl.ds(sid * RPT, RPT)],
                    out_hbm.at[pl.ds(cid * NPAD + sid * RPT, RPT)])


RW = NPAD // NW


def _fuse_body(F, relu, acc_hbm, hs_hbm, deg_hbm, b_hbm, batch_hbm,
               hout_hbm, maxout_hbm,
               acc0_v, acc1_v, hs_v, dinv_v, dg1_v, batch_v, b_v, maxtab):
    G = F // 16
    cid = lax.axis_index("c")
    sid = lax.axis_index("s")
    wid = sid * NC + cid
    base = wid * RW
    pltpu.sync_copy(acc_hbm.at[pl.ds(base, RW)], acc0_v)
    pltpu.sync_copy(acc_hbm.at[pl.ds(NPAD + base, RW)], acc1_v)
    pltpu.sync_copy(hs_hbm.at[pl.ds(base, RW)], hs_v)
    pltpu.sync_copy(deg_hbm.at[pl.ds(base, RW)], dinv_v)
    pltpu.sync_copy(deg_hbm.at[pl.ds(NPAD + base, RW)], dg1_v)
    pltpu.sync_copy(batch_hbm.at[pl.ds(base, RW)], batch_v)
    pltpu.sync_copy(b_hbm, b_v)

    def drow(r, c):
        sl = pl.ds(r * 16, 16)
        dinv_v[sl] = _dinv16(dinv_v[sl], dg1_v[sl])
        return c
    lax.fori_loop(0, RW // 16, drow, 0)

    def mrow(r, c):
        for g in range(G):
            maxtab[r, pl.ds(g * 16, 16)] = jnp.full((16,), _NEG_INF,
                                                    jnp.float32)
        return c
    lax.fori_loop(0, B, mrow, 0)

    def grp(g, c):
        b16 = batch_v[pl.ds(g * 16, 16)]
        d16 = dinv_v[pl.ds(g * 16, 16)]
        for u in range(16):
            r = g * 16 + u
            bid = b16[u]
            dv = d16[u]
            dd = dv * dv
            bidc = jnp.maximum(bid, 0)
            valid = bid >= 0
            for g2 in range(G):
                sl = pl.ds(g2 * 16, 16)
                v = (acc0_v[r, sl] + acc1_v[r, sl] + dd * hs_v[r, sl]
                     + b_v[sl])
                if relu:
                    v = jnp.maximum(v, 0.0)
                acc0_v[r, sl] = v
                mv = jnp.where(valid, v, _NEG_INF)
                maxtab[bidc, sl] = jnp.maximum(maxtab[bidc, sl], mv)
        return c
    lax.fori_loop(0, RW // 16, grp, 0)
    pltpu.sync_copy(acc0_v, hout_hbm.at[pl.ds(base, RW)])
    pltpu.sync_copy(maxtab, maxout_hbm.at[pl.ds(wid * B, B)])


@functools.cache
def _make_fuse_call(F, relu):
    return pl.kernel(
        functools.partial(_fuse_body, F, relu),
        out_type=[
            jax.ShapeDtypeStruct((NPAD, F), jnp.float32),
            jax.ShapeDtypeStruct((NW * B, F), jnp.float32),
        ],
        mesh=plsc.VectorSubcoreMesh(**_MESH),
        compiler_params=_SC_PARAMS,
        scratch_types=[
            pltpu.VMEM((RW, F), jnp.float32),
            pltpu.VMEM((RW, F), jnp.float32),
            pltpu.VMEM((RW, F), jnp.float32),
            pltpu.VMEM((RW,), jnp.float32),
            pltpu.VMEM((RW,), jnp.float32),
            pltpu.VMEM((RW,), jnp.int32),
            pltpu.VMEM((F,), jnp.float32),
            pltpu.VMEM((B, F), jnp.float32),
        ],
    )


@functools.cache
def _make_mp_call(F):
    return pl.kernel(
        functools.partial(_mp_body, F),
        out_type=jax.ShapeDtypeStruct((2 * NPAD, F), jnp.float32),
        mesh=plsc.VectorSubcoreMesh(**_MESH),
        compiler_params=_SC_PARAMS,
        scratch_types=[
            pltpu.VMEM((EPW,), jnp.int32),
            pltpu.VMEM((NCHUNK, K), jnp.int32),
            pltpu.VMEM((EPW,), jnp.float32),
            pltpu.VMEM((EPW,), jnp.float32),
            pltpu.VMEM((NPAD,), jnp.float32),
            pltpu.VMEM((RPT,), jnp.float32),
            pltpu.VMEM((RPT,), jnp.float32),
            pltpu.VMEM((K, F), jnp.float32),
            pltpu.VMEM((K, F), jnp.float32),
            pltpu.VMEM((K, F), jnp.float32),
            pltpu.VMEM_SHARED((NPAD, F), jnp.float32),
            pltpu.VMEM_SHARED((NPAD,), jnp.float32),
            pltpu.SemaphoreType.DMA,
            pltpu.SemaphoreType.DMA,
            pltpu.SemaphoreType.DMA,
            pltpu.SemaphoreType.DMA,
            pltpu.SemaphoreType.DMA,
            pltpu.SemaphoreType.DMA,
        ],
    )


def _matT(a, w):
    return lax.dot_general(a, w, (((1,), (1,)), ((), ())),
                           preferred_element_type=jnp.float32)


def _lrelu(x):
    return jnp.where(x > 0, x, 0.01 * x)


def _tc1_body(x_ref, w1_ref, hs_ref):
    hs_ref[pl.ds(0, N), :] = _matT(x_ref[...], w1_ref[...])
    hs_ref[pl.ds(N, NPAD - N), :] = jnp.zeros((NPAD - N, NHID), jnp.float32)


def _dense_body(ph1_ref, ph0_ref, eps_ref,
                e0w_ref, e0b_ref, e1w_ref, e1b_ref, d0w_ref, d0b_ref,
                d1w_ref, d1b_ref, ct0w_ref, ct0b_ref, bn1g_ref, bn1b_ref,
                bn2g_ref, bn2b_ref, ct1w_ref, ct1b_ref,
                mu_ref, land_ref, ldec_ref, betti_ref):
    emb = _lrelu(_matT(ph1_ref[...], e0w_ref[...]) + e0b_ref[...])
    mu = _matT(emb, e1w_ref[...]) + e1b_ref[...]
    mu_ref[...] = mu
    std = jnp.exp(0.5 * mu)
    land = eps_ref[...] * std + mu
    land_ref[...] = land
    dec = _lrelu(_matT(land, d0w_ref[...]) + d0b_ref[...])
    ld = _matT(dec, d1w_ref[...]) + d1b_ref[...]
    ldec_ref[...] = 1.0 / (1.0 + jnp.exp(-ld))

    bc = _matT(ph0_ref[...], ct0w_ref[...]) + ct0b_ref[...]
    bc = bc * _BN_SCALE * bn1g_ref[...] + bn1b_ref[...]
    bc = _lrelu(bc)
    bc = bc * _BN_SCALE * bn2g_ref[...] + bn2b_ref[...]
    betti_ref[...] = _matT(bc, ct1w_ref[...]) + ct1b_ref[...]


def _seg_mean(h, batch):
    maskf = (batch == lax.broadcasted_iota(jnp.int32, (N, B), 1)
             ).astype(jnp.float32)
    sums = lax.dot_general(maskf, h, (((0,), (0,)), ((), ())),
                           preferred_element_type=jnp.float32)
    cnt = lax.dot_general(maskf, jnp.ones((N, 1), jnp.float32),
                          (((0,), (0,)), ((), ())),
                          preferred_element_type=jnp.float32)
    return sums / jnp.maximum(cnt, 1.0)


def _maxcomb(parts_ref):
    mx = parts_ref[pl.ds(0, B), :]
    for t in range(1, NW):
        mx = jnp.maximum(mx, parts_ref[pl.ds(t * B, B), :])
    return mx


def _tc2_body(h1_ref, w2_ref, batch_ref, max1_ref,
              x1max_ref, x1mean_ref, h1s_ref):
    h1 = h1_ref[...]
    x1max_ref[...] = _maxcomb(max1_ref)
    x1mean_ref[...] = _seg_mean(h1[:N], batch_ref[...])
    h1s_ref[...] = _matT(h1, w2_ref[...])


def _tc3_body(h2_ref, batch_ref, max2_ref, x1max_ref, x1mean_ref,
              land_ref, betti_ref,
              m1wa_ref, m1wb_ref, m1wc_ref, m1wd_ref, m1we_ref, m1wf_ref,
              g0a_ref, g0b_ref, g0c_ref, g0d_ref, g0e_ref, g0f_ref,
              b0a_ref, b0b_ref, b0c_ref, b0d_ref, b0e_ref, b0f_ref,
              m1b_ref, bn1g_ref, bn1b_ref, m2w_ref, m2b_ref,
              clw_ref, clb_ref, cls_ref):
    h2 = h2_ref[...]
    x2max = _maxcomb(max2_ref)
    x2mean = _seg_mean(h2[:N], batch_ref[...])
    pieces = [
        (x1max_ref[...], m1wa_ref, g0a_ref, b0a_ref),
        (x1mean_ref[...], m1wb_ref, g0b_ref, b0b_ref),
        (x2max, m1wc_ref, g0c_ref, b0c_ref),
        (x2mean, m1wd_ref, g0d_ref, b0d_ref),
        (land_ref[...], m1we_ref, g0e_ref, b0e_ref),
        (betti_ref[...], m1wf_ref, g0f_ref, b0f_ref),
    ]
    f = m1b_ref[...]
    for val, wref, gref, bref in pieces:
        v = val * _BN_SCALE * gref[...] + bref[...]
        f = f + _matT(v, wref[...])
    f = jnp.maximum(f, 0.0)
    f = f * _BN_SCALE * bn1g_ref[...] + bn1b_ref[...]
    f = jnp.maximum(_matT(f, m2w_ref[...]) + m2b_ref[...], 0.0)
    logits = _matT(f, clw_ref[...]) + clb_ref[...]
    m = jnp.max(logits, axis=1, keepdims=True)
    lse = m + jnp.log(jnp.sum(jnp.exp(logits - m), axis=1, keepdims=True))
    cls_ref[...] = logits - lse


def _row2(v):
    return v.reshape(1, -1)


def kernel(x, edge_index, batch, edge_attr, PH1_feat, PH0_feat, params,
           vae_eps):
    p = params
    row = edge_index[0].reshape(NW, EPW)
    col = edge_index[1].reshape(NW, NCHUNK, K)
    w = edge_attr.reshape(NW, EPW)

    deg2 = _deg_call()(col, w)

    tc1 = pl.pallas_call(
        _tc1_body,
        out_shape=[
            jax.ShapeDtypeStruct((NPAD, NHID), jnp.float32),
        ],
    )
    (hs,) = tc1(x, p['c1_W'])

    dense = pl.pallas_call(
        _dense_body,
        out_shape=[
            jax.ShapeDtypeStruct((B, 16), jnp.float32),
            jax.ShapeDtypeStruct((B, 16), jnp.float32),
            jax.ShapeDtypeStruct((B, 1000), jnp.float32),
            jax.ShapeDtypeStruct((B, 32), jnp.float32),
        ],
    )
    mu, land, ldec, betti = dense(
        PH1_feat, PH0_feat, vae_eps,
        p['e0_W'], _row2(p['e0_b']), p['e1_W'], _row2(p['e1_b']),
        p['d0_W'], _row2(p['d0_b']), p['d1_W'], _row2(p['d1_b']),
        p['ct0_W'], _row2(p['ct0_b']), _row2(p['ct_bn1_g']),
        _row2(p['ct_bn1_b']), _row2(p['ct_bn2_g']), _row2(p['ct_bn2_b']),
        p['ct1_W'], _row2(p['ct1_b']))

    batchp = jnp.concatenate(
        [batch, jnp.full((NPAD - N,), -1, jnp.int32)])
    batch2 = batch.reshape(N, 1)

    acc1 = _make_mp_call(NHID)(hs, row, col, w, deg2)
    h1, max1 = _make_fuse_call(NHID, True)(
        acc1, hs, deg2, p['c1_b'], batchp)

    tc2 = pl.pallas_call(
        _tc2_body,
        out_shape=[
            jax.ShapeDtypeStruct((B, NHID), jnp.float32),
            jax.ShapeDtypeStruct((B, NHID), jnp.float32),
            jax.ShapeDtypeStruct((NPAD, OUTF), jnp.float32),
        ],
    )
    x1max, x1mean, h1s = tc2(h1, p['c2_W'], batch2, max1)

    acc2 = _make_mp_call(OUTF)(h1s, row, col, w, deg2)
    h2, max2 = _make_fuse_call(OUTF, False)(
        acc2, h1s, deg2, p['c2_b'], batchp)

    m1 = p['m1_W']
    g0 = p['m_bn0_g']
    b0 = p['m_bn0_b']
    splits = [0, 64, 128, 144, 160, 176, 208]
    m1w = [m1[:, splits[i]:splits[i + 1]] for i in range(6)]
    g0s = [_row2(g0[splits[i]:splits[i + 1]]) for i in range(6)]
    b0s = [_row2(b0[splits[i]:splits[i + 1]]) for i in range(6)]

    tc3 = pl.pallas_call(
        _tc3_body,
        out_shape=[jax.ShapeDtypeStruct((B, 2), jnp.float32)],
    )
    (cls,) = tc3(h2, batch2, max2, x1max, x1mean, land, betti,
                 *m1w, *g0s, *b0s,
                 _row2(p['m1_b']), _row2(p['m_bn1_g']), _row2(p['m_bn1_b']),
                 p['m2_W'], _row2(p['m2_b']), p['cl_W'], _row2(p['cl_b']))

    return (cls, mu, mu, land, ldec, betti)
